# baseline (speedup 1.0000x reference)
import functools

import jax
import jax.numpy as jnp
from jax import lax
from jax.experimental import pallas as pl
from jax.experimental.pallas import tpu as pltpu
from jax.experimental.pallas import tpu_sc as plsc

_INFO = plsc.get_sparse_core_info()
_NC, _NS = _INFO.num_cores, _INFO.num_subcores
_NW = _NC * _NS

_CHUNK = 200
_DP = 128


def _pad_table(table, dp):
  V, D = table.shape
  blk = 2000
  assert V % blk == 0

  def body(t_ref, o_ref):
    o_ref[...] = jnp.concatenate(
        [t_ref[...], jnp.zeros((blk, dp - D), jnp.float32)], axis=1
    )

  return pl.pallas_call(
      body,
      grid=(V // blk,),
      in_specs=[pl.BlockSpec((blk, D), lambda i: (i, 0))],
      out_specs=pl.BlockSpec((blk, dp), lambda i: (i, 0)),
      out_shape=jax.ShapeDtypeStruct((V, dp), jnp.float32),
  )(table)


def _make_gather(V: int, D: int, Bq: int, S: int):
  B = Bq * S
  assert S == _CHUNK and B % (_NW * 2 * _CHUNK) == 0
  b_per_w = B // _NW
  bq_per_w = Bq // _NW
  n_chunks = b_per_w // _CHUNK
  n_pairs = n_chunks // 2
  mesh = plsc.VectorSubcoreMesh(core_axis_name="c", subcore_axis_name="s")

  @functools.partial(
      pl.kernel,
      mesh=mesh,
      out_type=jax.ShapeDtypeStruct((Bq, S, D), jnp.float32),
      scratch_types=[
          pltpu.VMEM((b_per_w,), jnp.int32),
          pltpu.VMEM((_CHUNK, _DP), jnp.float32),
          pltpu.VMEM((_CHUNK, _DP), jnp.float32),
          pltpu.VMEM((_CHUNK, D), jnp.float32),
          pltpu.VMEM((_CHUNK, D), jnp.float32),
          pltpu.SemaphoreType.DMA,
          pltpu.SemaphoreType.DMA,
          pltpu.SemaphoreType.DMA,
          pltpu.SemaphoreType.DMA,
      ],
  )
  def gather_kernel(table_hbm, idx_hbm, out_hbm, idx_all, w0, w1, n0, n1,
                    gs0, gs1, ss0, ss1):
    wid = lax.axis_index("s") * _NC + lax.axis_index("c")
    base = wid * b_per_w
    base_b = wid * bq_per_w
    wide = (w0, w1)
    narrow = (n0, n1)
    gsem = (gs0, gs1)
    ssem = (ss0, ss1)

    pltpu.sync_copy(idx_hbm.at[pl.ds(base, b_per_w)], idx_all)

    def gather_start(g, b):
      pltpu.async_copy(
          table_hbm.at[idx_all.at[pl.ds(g * _CHUNK, _CHUNK)]], wide[b], gsem[b]
      )

    def store_start(g, b):
      pltpu.async_copy(narrow[b], out_hbm.at[base_b + g], ssem[b])

    def gather_wait(b):
      pltpu.make_async_copy(
          table_hbm.at[pl.ds(0, _CHUNK)], wide[b], gsem[b]
      ).wait()

    def store_wait(b):
      pltpu.make_async_copy(narrow[b], out_hbm.at[base_b], ssem[b]).wait()

    def repack(b):
      src = wide[b]
      dst = narrow[b]

      def row(j, carry):
        for c in range(D // 16):
          dst[j, pl.ds(16 * c, 16)] = src[j, pl.ds(16 * c, 16)]
        return carry

      lax.fori_loop(0, _CHUNK, row, 0)

    def turn(g, b, with_store_wait, with_next_gather):
      gather_wait(b)
      if with_store_wait:
        store_wait(b)
      repack(b)
      if with_next_gather:
        gather_start(g + 2, b)
      store_start(g, b)

    gather_start(0, 0)
    gather_start(1, 1)
    turn(0, 0, False, True)
    turn(1, 1, False, True)

    def pair_body(r, carry):
      g0 = 2 * r
      turn(g0, 0, True, True)
      turn(g0 + 1, 1, True, True)
      return carry

    lax.fori_loop(1, n_pairs - 1, pair_body, 0)

    turn(n_chunks - 2, 0, True, False)
    turn(n_chunks - 1, 1, True, False)
    store_wait(0)
    store_wait(1)

  return gather_kernel


def kernel(tokens, table):
  Bq, S = tokens.shape
  V, D = table.shape
  idx = tokens.reshape(-1).astype(jnp.int32)
  table_p = _pad_table(table, _DP)
  return _make_gather(V, D, Bq, S)(table_p, idx)


# speedup vs baseline: 1.1427x; 1.1427x over previous
import functools

import jax
import jax.numpy as jnp
from jax import lax
from jax.experimental import pallas as pl
from jax.experimental.pallas import tpu as pltpu
from jax.experimental.pallas import tpu_sc as plsc

_INFO = plsc.get_sparse_core_info()
_NC, _NS = _INFO.num_cores, _INFO.num_subcores
_NW = _NC * _NS

_CHUNK = 400
_DP = 128


def _pad_table(table, dp):
  V, D = table.shape
  blk = 4000
  assert V % blk == 0

  def body(t_ref, o_ref):
    o_ref[:, :D] = t_ref[...]

  return pl.pallas_call(
      body,
      grid=(V // blk,),
      in_specs=[pl.BlockSpec((blk, D), lambda i: (i, 0))],
      out_specs=pl.BlockSpec((blk, dp), lambda i: (i, 0)),
      out_shape=jax.ShapeDtypeStruct((V, dp), jnp.float32),
  )(table)


def _make_gather(V: int, B: int):
  assert B % (_NW * 2 * _CHUNK) == 0
  b_per_w = B // _NW
  n_chunks = b_per_w // _CHUNK
  n_rounds = n_chunks // 2
  mesh = plsc.VectorSubcoreMesh(core_axis_name="c", subcore_axis_name="s")

  @functools.partial(
      pl.kernel,
      mesh=mesh,
      out_type=jax.ShapeDtypeStruct((B, _DP), jnp.float32),
      scratch_types=[
          pltpu.VMEM((b_per_w,), jnp.int32),
          pltpu.VMEM((_CHUNK, _DP), jnp.float32),
          pltpu.VMEM((_CHUNK, _DP), jnp.float32),
          pltpu.SemaphoreType.DMA,
          pltpu.SemaphoreType.DMA,
          pltpu.SemaphoreType.DMA,
          pltpu.SemaphoreType.DMA,
      ],
  )
  def gather_kernel(table_hbm, idx_hbm, out_hbm, idx_all, rows0, rows1,
                    gs0, gs1, ss0, ss1):
    wid = lax.axis_index("s") * _NC + lax.axis_index("c")
    base = wid * b_per_w
    rows = (rows0, rows1)
    gsem = (gs0, gs1)
    ssem = (ss0, ss1)

    pltpu.sync_copy(idx_hbm.at[pl.ds(base, b_per_w)], idx_all)

    def gather_start(g, b):
      pltpu.async_copy(
          table_hbm.at[idx_all.at[pl.ds(g * _CHUNK, _CHUNK)]], rows[b], gsem[b]
      )

    def store_start(g, b):
      pltpu.async_copy(
          rows[b], out_hbm.at[pl.ds(base + g * _CHUNK, _CHUNK)], ssem[b]
      )

    def gather_wait(b):
      pltpu.make_async_copy(
          table_hbm.at[pl.ds(0, _CHUNK)], rows[b], gsem[b]
      ).wait()

    def store_wait(b):
      pltpu.make_async_copy(
          rows[b], out_hbm.at[pl.ds(base, _CHUNK)], ssem[b]
      ).wait()

    gather_start(0, 0)
    gather_start(1, 1)
    gather_wait(0)
    store_start(0, 0)

    def round_body(r, carry):
      g0 = 2 * r
      store_wait(0)
      gather_start(g0, 0)
      gather_wait(1)
      store_start(g0 - 1, 1)
      store_wait(1)
      gather_start(g0 + 1, 1)
      gather_wait(0)
      store_start(g0, 0)
      return carry

    lax.fori_loop(1, n_rounds, round_body, 0)

    gather_wait(1)
    store_start(n_chunks - 1, 1)
    store_wait(0)
    store_wait(1)

  return gather_kernel


def kernel(tokens, table):
  Bq, S = tokens.shape
  V, D = table.shape
  idx = tokens.reshape(-1).astype(jnp.int32)
  table_p = _pad_table(table, _DP)
  out = _make_gather(V, Bq * S)(table_p, idx)
  return out[:, :D].reshape(Bq, S, D)
